# Initial kernel scaffold; baseline (speedup 1.0000x reference)
import jax
import jax.numpy as jnp
from jax.experimental import pallas as pl


def kernel(x, params):
    raise NotImplementedError("write your pallas kernel here")


# speedup vs baseline: 5.0932x; 5.0932x over previous
import functools

import numpy as np
import jax
import jax.numpy as jnp
from jax import lax
from jax.experimental import pallas as pl

B = 8
N0 = 1024
IN_DIM = 6
NBLOCKS = 4
KNN = 16
DM = 128
NPTS = [N0 // 4 ** (i + 1) for i in range(NBLOCKS)]
CHS = [32 * 2 ** (i + 1) for i in range(NBLOCKS)]


def _mm(a, b):
    return lax.dot_general(a, b, (((1,), (1,)), ((), ())))


def _mmb(a, b):
    return lax.dot_general(a.astype(jnp.bfloat16), b.astype(jnp.bfloat16),
                           (((1,), (1,)), ((), ())),
                           preferred_element_type=jnp.float32)


def _mm_nt(a, b):
    return lax.dot_general(a, b, (((1,), (0,)), ((), ())),
                           precision=lax.Precision.HIGHEST)


def _wspec(shape):
    nd = len(shape)
    return pl.BlockSpec(shape, lambda *args: (0,) * nd)


def _embed_body(x_ref, aw_ref, ab_ref, bw_ref, bb_ref, o_ref):
    h = jnp.maximum(_mmb(x_ref[0], aw_ref[...]) + ab_ref[...], 0.0)
    o_ref[0] = _mmb(h, bw_ref[...]) + bb_ref[...]


def _embed(x, pa, pb):
    aw, ab = pa['w'], pa['b'].reshape(1, -1)
    bw, bb = pb['w'], pb['b'].reshape(1, -1)
    return pl.pallas_call(
        _embed_body,
        grid=(B,),
        in_specs=[pl.BlockSpec((1, N0, IN_DIM), lambda b: (b, 0, 0)),
                  _wspec(aw.shape), _wspec(ab.shape),
                  _wspec(bw.shape), _wspec(bb.shape)],
        out_specs=pl.BlockSpec((1, N0, 32), lambda b: (b, 0, 0)),
        out_shape=jax.ShapeDtypeStruct((B, N0, 32), jnp.float32),
    )(x, aw, ab, bw, bb)


def _knn_body(src_ref, dt_ref, o_ref, *, m, n, keff):
    sx = src_ref[0][:, 0:1]
    sy = src_ref[0][:, 1:2]
    sz = src_ref[0][:, 2:3]
    dx = dt_ref[0, 0:1, :]
    dy = dt_ref[0, 1:2, :]
    dz = dt_ref[0, 2:3, :]
    s16 = src_ref[0].astype(jnp.bfloat16)
    d16 = dt_ref[0].astype(jnp.bfloat16)
    prod = lax.dot_general(s16, d16, (((1,), (0,)), ((), ())),
                           preferred_element_type=jnp.float32)
    ssq = (sx * sx + sy * sy) + sz * sz
    dsq = (dx * dx + dy * dy) + dz * dz
    dist = (-2.0 * prod + ssq) + dsq
    lane = lax.broadcasted_iota(jnp.int32, (m, n), 1)
    kio = lax.broadcasted_iota(jnp.int32, (m, keff), 1)
    out = jnp.zeros((m, keff), jnp.int32)
    for k in range(keff):
        mn = jnp.min(dist, axis=1, keepdims=True)
        idxv = jnp.min(jnp.where(dist == mn, lane, n), axis=1, keepdims=True)
        out = jnp.where(kio == k, idxv, out)
        dist = jnp.where(lane == idxv, jnp.inf, dist)
    o_ref[0] = out


def _knn(src, dst):
    m = src.shape[1]
    n = dst.shape[1]
    keff = min(KNN, n)
    dstT = jnp.transpose(dst, (0, 2, 1))
    body = functools.partial(_knn_body, m=m, n=n, keff=keff)
    return pl.pallas_call(
        body,
        grid=(B,),
        in_specs=[pl.BlockSpec((1, m, 3), lambda b: (b, 0, 0)),
                  pl.BlockSpec((1, 3, n), lambda b: (b, 0, 0))],
        out_specs=pl.BlockSpec((1, m, keff), lambda b: (b, 0, 0)),
        out_shape=jax.ShapeDtypeStruct((B, m, keff), jnp.int32),
    )(src, dstT)


def _fps_body(xt_ref, o_ref, *, n, npoint):
    xs = xt_ref[0]
    ys = xt_ref[1]
    zs = xt_ref[2]
    lane = lax.broadcasted_iota(jnp.int32, (B, n), 1)
    pio = lax.broadcasted_iota(jnp.int32, (B, npoint), 1)

    o_ref[...] = jnp.zeros((B, npoint), jnp.int32)

    def body(i, st):
        dist, far = st
        o_ref[...] = jnp.where(pio == i, jnp.broadcast_to(far, (B, npoint)),
                               o_ref[...])
        mask = lane == far
        cx = jnp.sum(jnp.where(mask, xs, 0.0), axis=1, keepdims=True)
        cy = jnp.sum(jnp.where(mask, ys, 0.0), axis=1, keepdims=True)
        cz = jnp.sum(jnp.where(mask, zs, 0.0), axis=1, keepdims=True)
        dx = xs - cx
        dy = ys - cy
        dz = zs - cz
        d = (dx * dx + dy * dy) + dz * dz
        dist = jnp.minimum(dist, d)
        mx = jnp.max(dist, axis=1, keepdims=True)
        far = jnp.min(jnp.where(dist == mx, lane, n), axis=1, keepdims=True)
        return dist, far

    init = (jnp.full((B, n), 1e10, jnp.float32),
            jnp.zeros((B, 1), jnp.int32))
    lax.fori_loop(0, npoint, body, init)


def _fps(xyz, npoint):
    n = xyz.shape[1]
    xt = jnp.transpose(xyz, (2, 0, 1))
    body = functools.partial(_fps_body, n=n, npoint=npoint)
    return pl.pallas_call(
        body,
        out_shape=jax.ShapeDtypeStruct((B, npoint), jnp.int32),
    )(xt)


def _gidx_body(cent_ref, xyz_ref, o_ref, *, n, m):
    c = cent_ref[0]
    rio = lax.broadcasted_iota(jnp.int32, (n, m), 0)
    oht = (rio == c).astype(jnp.float32)
    cols = []
    for ch in range(3):
        xc = xyz_ref[0][:, ch:ch + 1]
        cols.append(jnp.sum(oht * xc, axis=0, keepdims=True))
    o_ref[0] = jnp.concatenate(cols, axis=0)


def _gather_xyz(cent, xyz):
    m = cent.shape[1]
    n = xyz.shape[1]
    cent3 = cent.reshape(B, 1, m)
    body = functools.partial(_gidx_body, n=n, m=m)
    out = pl.pallas_call(
        body,
        grid=(B,),
        in_specs=[pl.BlockSpec((1, 1, m), lambda b: (b, 0, 0)),
                  pl.BlockSpec((1, n, 3), lambda b: (b, 0, 0))],
        out_specs=pl.BlockSpec((1, 3, m), lambda b: (b, 0, 0)),
        out_shape=jax.ShapeDtypeStruct((B, 3, m), jnp.float32),
    )(cent3, xyz)
    return jnp.transpose(out, (0, 2, 1))


def _tables_body(f_ref, fw_ref, fb_ref, qw_ref, kw_ref, vw_ref,
                 q_ref, k_ref, v_ref):
    h = _mmb(f_ref[0], fw_ref[...]) + fb_ref[...]
    q_ref[0] = _mmb(h, qw_ref[...])
    k_ref[0] = _mmb(h, kw_ref[...])
    v_ref[0] = _mmb(h, vw_ref[...])


def _tables(feat, p):
    n = feat.shape[1]
    dp = feat.shape[2]
    fw, fb = p['fc1']['w'], p['fc1']['b'].reshape(1, -1)
    qw, kw, vw = p['wq']['w'], p['wk']['w'], p['wv']['w']
    shp = jax.ShapeDtypeStruct((B, n, DM), jnp.float32)
    return pl.pallas_call(
        _tables_body,
        grid=(B,),
        in_specs=[pl.BlockSpec((1, n, dp), lambda b: (b, 0, 0)),
                  _wspec(fw.shape), _wspec(fb.shape), _wspec(qw.shape),
                  _wspec(kw.shape), _wspec(vw.shape)],
        out_specs=tuple(pl.BlockSpec((1, n, DM), lambda b: (b, 0, 0))
                        for _ in range(3)),
        out_shape=(shp, shp, shp),
    )(feat, fw, fb, qw, kw, vw)


def _attn_body(q_ref, pre_ref, idx_ref, k_ref, v_ref, xyz_ref,
               d1w_ref, d1b_ref, d2w_ref, d2b_ref, g1w_ref, g1b_ref, g2w_ref,
               g2b_ref, f2w_ref, f2b_ref, o_ref, *, t, keff, n):
    ti = pl.program_id(1)
    idx = idx_ref[0]
    io3 = lax.broadcasted_iota(jnp.int32, (t, keff, n), 2)
    oh = (idx[:, :, None] == io3).astype(jnp.float32).reshape(t * keff, n)
    kg = _mm_nt(oh, k_ref[0])
    vg = _mm_nt(oh, v_ref[0])
    xg = _mm_nt(oh, xyz_ref[0])
    x_t = xyz_ref[0, pl.ds(ti * t, t), :]
    xrep = jnp.broadcast_to(x_t[:, None, :], (t, keff, 3)).reshape(t * keff, 3)
    qrep = jnp.broadcast_to(q_ref[0][:, None, :], (t, keff, DM)).reshape(t * keff, DM)
    pe1 = jnp.maximum(_mmb(xrep - xg, d1w_ref[...]) + d1b_ref[...], 0.0)
    pos = _mmb(pe1, d2w_ref[...]) + d2b_ref[...]
    a1 = jnp.maximum(_mmb(qrep - kg + pos, g1w_ref[...]) + g1b_ref[...], 0.0)
    at = _mmb(a1, g2w_ref[...]) + g2b_ref[...]
    s3 = (at / np.sqrt(float(DM))).reshape(t, keff, DM)
    mx = jnp.max(s3, axis=1, keepdims=True)
    e = jnp.exp(s3 - mx)
    den = e[:, 0, :]
    for k in range(1, keff):
        den = den + e[:, k, :]
    sm = e / den[:, None, :]
    vp = vg.reshape(t, keff, DM) + pos.reshape(t, keff, DM)
    pr = sm * vp
    res = pr[:, 0, :]
    for k in range(1, keff):
        res = res + pr[:, k, :]
    o_ref[0] = _mmb(res, f2w_ref[...]) + f2b_ref[...] + pre_ref[0]


def _attn(q, pre, idx, ktab, vtab, xyz, p, t):
    n = q.shape[1]
    dp = pre.shape[2]
    keff = idx.shape[2]
    nt = n // t
    d1w, d1b = p['d1']['w'], p['d1']['b'].reshape(1, -1)
    d2w, d2b = p['d2']['w'], p['d2']['b'].reshape(1, -1)
    g1w, g1b = p['g1']['w'], p['g1']['b'].reshape(1, -1)
    g2w, g2b = p['g2']['w'], p['g2']['b'].reshape(1, -1)
    f2w, f2b = p['fc2']['w'], p['fc2']['b'].reshape(1, -1)
    body = functools.partial(_attn_body, t=t, keff=keff, n=n)
    return pl.pallas_call(
        body,
        grid=(B, nt),
        in_specs=[pl.BlockSpec((1, t, DM), lambda b, i: (b, i, 0)),
                  pl.BlockSpec((1, t, dp), lambda b, i: (b, i, 0)),
                  pl.BlockSpec((1, t, keff), lambda b, i: (b, i, 0)),
                  pl.BlockSpec((1, n, DM), lambda b, i: (b, 0, 0)),
                  pl.BlockSpec((1, n, DM), lambda b, i: (b, 0, 0)),
                  pl.BlockSpec((1, n, 3), lambda b, i: (b, 0, 0)),
                  _wspec(d1w.shape), _wspec(d1b.shape),
                  _wspec(d2w.shape), _wspec(d2b.shape),
                  _wspec(g1w.shape), _wspec(g1b.shape), _wspec(g2w.shape),
                  _wspec(g2b.shape), _wspec(f2w.shape), _wspec(f2b.shape)],
        out_specs=pl.BlockSpec((1, t, dp), lambda b, i: (b, i, 0)),
        out_shape=jax.ShapeDtypeStruct((B, n, dp), jnp.float32),
    )(q, pre, idx, ktab, vtab, xyz, d1w, d1b, d2w, d2b, g1w, g1b, g2w, g2b,
      f2w, f2b)


def _tblock(xyz, feat, p, t):
    idx = _knn(xyz, xyz)
    q, ktab, vtab = _tables(feat, p)
    return _attn(q, feat, idx, ktab, vtab, xyz, p, t)


def _td_body(xyz_ref, nxyz_ref, pts_ref, idx_ref, c1w_ref, c1b_ref, g1_ref,
             b1_ref, c2w_ref, c2b_ref, g2_ref, b2_ref, o_ref, *, m, n, cp, c):
    nk = m * KNN
    ntot = float(B * nk)
    h1 = []
    s1 = jnp.zeros((1, c), jnp.float32)
    for b in range(B):
        tab = jnp.concatenate([xyz_ref[b], pts_ref[b]], axis=1)
        idx = idx_ref[b]
        io3 = lax.broadcasted_iota(jnp.int32, (m, KNN, n), 2)
        oh = (idx[:, :, None] == io3).astype(jnp.float32).reshape(nk, n)
        g = _mm_nt(oh, tab)
        nrep = jnp.broadcast_to(nxyz_ref[b][:, None, :],
                                (m, KNN, 3)).reshape(nk, 3)
        sub = jnp.concatenate([nrep, jnp.zeros((nk, cp), jnp.float32)], axis=1)
        hb = _mmb(g - sub, c1w_ref[...]) + c1b_ref[...]
        h1.append(hb)
        s1 = s1 + jnp.sum(hb, axis=0, keepdims=True)
    mean1 = s1 / ntot
    v1 = jnp.zeros((1, c), jnp.float32)
    for b in range(B):
        dlt = h1[b] - mean1
        v1 = v1 + jnp.sum(dlt * dlt, axis=0, keepdims=True)
    den1 = jnp.sqrt(v1 / ntot + 1e-5)
    h2 = []
    s2 = jnp.zeros((1, c), jnp.float32)
    for b in range(B):
        hn = jnp.maximum((h1[b] - mean1) / den1 * g1_ref[...] + b1_ref[...],
                         0.0)
        hb = _mmb(hn, c2w_ref[...]) + c2b_ref[...]
        h2.append(hb)
        s2 = s2 + jnp.sum(hb, axis=0, keepdims=True)
    mean2 = s2 / ntot
    v2 = jnp.zeros((1, c), jnp.float32)
    for b in range(B):
        dlt = h2[b] - mean2
        v2 = v2 + jnp.sum(dlt * dlt, axis=0, keepdims=True)
    den2 = jnp.sqrt(v2 / ntot + 1e-5)
    for b in range(B):
        hn = jnp.maximum((h2[b] - mean2) / den2 * g2_ref[...] + b2_ref[...],
                         0.0)
        o_ref[b] = jnp.max(hn.reshape(m, KNN, c), axis=1)


def _td(xyz, new_xyz, pts, idx, p, c):
    n = xyz.shape[1]
    m = new_xyz.shape[1]
    cp = pts.shape[2]
    c1w, c1b = p['conv1']['w'], p['conv1']['b'].reshape(1, -1)
    g1, b1 = p['bn1']['g'].reshape(1, -1), p['bn1']['b'].reshape(1, -1)
    c2w, c2b = p['conv2']['w'], p['conv2']['b'].reshape(1, -1)
    g2, b2 = p['bn2']['g'].reshape(1, -1), p['bn2']['b'].reshape(1, -1)
    body = functools.partial(_td_body, m=m, n=n, cp=cp, c=c)
    return pl.pallas_call(
        body,
        out_shape=jax.ShapeDtypeStruct((B, m, c), jnp.float32),
    )(xyz, new_xyz, pts, idx, c1w, c1b, g1, b1, c2w, c2b, g2, b2)


def kernel(x, params):
    x = jnp.asarray(x, jnp.float32)
    xyz = x[..., :3]
    h0 = _embed(x, params['fc1a'], params['fc1b'])
    pts = _tblock(xyz, h0, params['t1'], t=128)
    feats = [(xyz, pts)]
    for i in range(NBLOCKS):
        cent = _fps(xyz, NPTS[i])
        new_xyz = _gather_xyz(cent, xyz)
        idx = _knn(new_xyz, xyz)
        pts = _td(xyz, new_xyz, pts, idx, params['td'][i], CHS[i])
        xyz = new_xyz
        pts = _tblock(xyz, pts, params['tf'][i], t=NPTS[i])
        feats.append((xyz, pts))
    return pts, feats
